# baseline (speedup 1.0000x reference)
import dataclasses
import functools

import jax
import jax.numpy as jnp
from jax import lax
from jax.experimental import pallas as pl
from jax.experimental.pallas import tpu as pltpu
from jax.experimental.pallas import tpu_sc as plsc

_GAMMA = 24.0
_B = 4096
_NEG = 128
_D = 64
_L = 16
_NV = _D // _L
_NC = 2
_NS = 16
_NW = _NC * _NS
_BPW = _B // _NW
_CB = 4
_CROWS = _CB * _NEG
_NCHUNK = _BPW // _CB

_NENT = 1000000
_TCH = 384
_TNCH = 2604
_TTAIL = _NENT - _TNCH * _TCH
_TITER = 82


def _tr_body(et_hbm, s0, s1, s2, s3,
             t0, t1, ob00, ob01, ob02, ob03, ob10, ob11, ob12, ob13,
             tt, obt0, obt1, obt2, obt3,
             semi0, semi1, semw0, semw1):
  outs = (s0, s1, s2, s3)
  obt = (obt0, obt1, obt2, obt3)
  obs = ((ob00, ob01, ob02, ob03), (ob10, ob11, ob12, ob13))
  tb = (t0, t1)
  semi = (semi0, semi1)
  semw = (semw0, semw1)
  wid = lax.axis_index("s") * _NC + lax.axis_index("c")
  lane = lax.iota(jnp.int32, _L)
  lane16 = lane * _L

  def start_in(c, par):
    @pl.when(c < _TNCH)
    def _():
      pltpu.async_copy(et_hbm.at[:, pl.ds(c * _TCH, _TCH)], tb[par], semi[par])

  def wait_in(par):
    pltpu.make_async_copy(et_hbm.at[:, pl.ds(0, _TCH)], tb[par], semi[par]).wait()

  def start_out(c, par):
    for m in range(_NV):
      pltpu.async_copy(obs[par][m],
                       outs[m].at[pl.ds(c * _TCH * _L, _TCH * _L)], semw[par])

  def wait_out(par):
    for m in range(_NV):
      pltpu.make_async_copy(obs[par][m],
                            outs[m].at[pl.ds(0, _TCH * _L)], semw[par]).wait()

  def transpose_chunk(tbuf, obufs, width):
    @plsc.parallel_loop(0, width, step=_L, unroll=2)
    def _(e0):
      base = e0 * _L
      idxs = [lane16 + (base + i) for i in range(_L)]
      vs = [tbuf[i, pl.ds(e0, _L)] for i in range(_L)]
      for m in range(_NV):
        nxt = []
        for i in range(_L):
          if m + 1 < _NV:
            nxt.append(tbuf[(m + 1) * _L + i, pl.ds(e0, _L)])
          plsc.store_scatter(obufs[m], [idxs[i]], vs[i])
        vs = nxt

  start_in(wid, 0)
  start_in(wid + _NW, 1)

  @pl.loop(0, _TITER, step=2)
  def _(k):
    c0 = wid + k * _NW
    c1 = wid + (k + 1) * _NW

    @pl.when(c0 < _TNCH)
    def _():
      wait_in(0)

      @pl.when(k > 0)
      def _():
        wait_out(0)

      transpose_chunk(tb[0], obs[0], _TCH)
      start_in(c0 + 2 * _NW, 0)
      start_out(c0, 0)

    @pl.when(c1 < _TNCH)
    def _():
      wait_in(1)

      @pl.when(k > 0)
      def _():
        wait_out(1)

      transpose_chunk(tb[1], obs[1], _TCH)
      start_in(c1 + 2 * _NW, 1)
      start_out(c1, 1)

  @pl.when(wid + (_TITER - 2) * _NW < _TNCH)
  def _():
    wait_out(0)

  @pl.when(wid + (_TITER - 1) * _NW < _TNCH)
  def _():
    wait_out(1)

  @pl.when(wid == 0)
  def _():
    pltpu.async_copy(et_hbm.at[:, pl.ds(_TNCH * _TCH, _TTAIL)], tt,
                     semi0).wait()

    @pl.loop(0, _TTAIL, step=_L)
    def _(e0):
      base = e0 * _L
      idxs = [lane16 + (base + i) for i in range(_L)]
      vs = [tt[i, pl.ds(e0, _L)] for i in range(_L)]
      for m in range(_NV):
        nxt = []
        for i in range(_L):
          if m + 1 < _NV:
            nxt.append(tt[(m + 1) * _L + i, pl.ds(e0, _L)])
          plsc.store_scatter(obt[m], [idxs[i]], vs[i])
        vs = nxt

    for m in range(_NV):
      pltpu.sync_copy(obt[m],
                      outs[m].at[pl.ds(_TNCH * _TCH * _L, _TTAIL * _L)])


def _sc_body(e0, e1, e2, e3, rel_hbm, aidx_hbm, ridx_hbm, pidx_hbm, nidx_hbm,
             pos_hbm, neg_hbm,
             aidx_v, ridx_v, pidx_v,
             a0, a1, a2, a3, p0, p1, p2, p3, rrow_v, cent_v, pos_v,
             nidx0_v, nidx1_v,
             g00, g01, g02, g03, g10, g11, g12, g13, o0_v, o1_v,
             sem_a, sem_r, sem_p, sem0, sem1):
  slabs = (e0, e1, e2, e3)
  arow = (a0, a1, a2, a3)
  prow = (p0, p1, p2, p3)
  gbuf0 = (g00, g01, g02, g03)
  gbuf1 = (g10, g11, g12, g13)

  wid = lax.axis_index("s") * _NC + lax.axis_index("c")
  base = wid * _BPW
  nbase = wid * (_BPW * _NEG)

  pltpu.sync_copy(aidx_hbm.at[pl.ds(base, _BPW)], aidx_v)
  pltpu.sync_copy(ridx_hbm.at[pl.ds(base, _BPW)], ridx_v)
  pltpu.sync_copy(pidx_hbm.at[pl.ds(base, _BPW)], pidx_v)
  for m in range(_NV):
    pltpu.async_copy(slabs[m].at[aidx_v], arow[m], sem_a)
    pltpu.async_copy(slabs[m].at[pidx_v], prow[m], sem_p)
  cr = pltpu.async_copy(rel_hbm.at[ridx_v], rrow_v, sem_r)

  def start_gather(cidx, idxbuf, gbufs, sem):
    row_lo = base + cidx * _CB
    pltpu.sync_copy(nidx_hbm.at[pl.ds(row_lo, _CB)], idxbuf)
    for q in range(_CB):
      for m in range(_NV):
        pltpu.async_copy(slabs[m].at[idxbuf.at[q]],
                         gbufs[m].at[pl.ds(q * _NEG, _NEG)], sem)

  def wait_gather(idxbuf, gbufs, sem):
    for q in range(_CB):
      for m in range(_NV):
        pltpu.make_async_copy(slabs[m].at[idxbuf.at[q]],
                              gbufs[m].at[pl.ds(q * _NEG, _NEG)], sem).wait()

  start_gather(0, nidx0_v, gbuf0, sem0)

  for m in range(_NV):
    pltpu.make_async_copy(slabs[m].at[aidx_v], arow[m], sem_a).wait()
    pltpu.make_async_copy(slabs[m].at[pidx_v], prow[m], sem_p).wait()
  cr.wait()

  lane = lax.iota(jnp.int32, _L)

  @pl.loop(0, _BPW // _L)
  def _(g):
    out = jnp.zeros((_L,), jnp.float32)
    for jj in range(_L):
      b = g * _L + jj
      acc = None
      for k in range(_NV):
        c = arow[k][b, pl.ds(0, _L)] + rrow_v[b, pl.ds(k * _L, _L)]
        cent_v[b, pl.ds(k * _L, _L)] = c
        d = jnp.abs(c - prow[k][b, pl.ds(0, _L)])
        acc = d if acc is None else acc + d
      out = jnp.where(lane == jj, jnp.sum(acc), out)
    pos_v[pl.ds(g * _L, _L)] = _GAMMA - out

  pltpu.sync_copy(pos_v, pos_hbm.at[pl.ds(base, _BPW)])

  lane2 = lax.iota(jnp.int32, _L)

  def compute_chunk(cidx, gbufs, obuf):
    for bb in range(_CB):
      brow = cidx * _CB + bb
      cvec = [cent_v[brow, pl.ds(k * _L, _L)] for k in range(_NV)]

      @pl.loop(0, _NEG // _L)
      def _(g):
        out = jnp.zeros((_L,), jnp.float32)
        for jj in range(_L):
          r = bb * _NEG + g * _L + jj
          ld = [gbufs[k][r, pl.ds(0, _L)] for k in range(_NV)]
          d = [jnp.abs(cvec[k] - ld[k]) for k in range(_NV)]
          out = jnp.where(lane2 == jj, jnp.sum((d[0] + d[1]) + (d[2] + d[3])),
                          out)
        obuf[pl.ds(bb * _NEG + g * _L, _L)] = _GAMMA - out

    pltpu.sync_copy(obuf, neg_hbm.at[pl.ds(nbase + cidx * _CROWS, _CROWS)])

  @pl.loop(0, _NCHUNK, step=2)
  def _(c):
    wait_gather(nidx0_v, gbuf0, sem0)
    start_gather(c + 1, nidx1_v, gbuf1, sem1)
    compute_chunk(c, gbuf0, o0_v)

    wait_gather(nidx1_v, gbuf1, sem1)

    @pl.when(c + 2 < _NCHUNK)
    def _():
      start_gather(c + 2, nidx0_v, gbuf0, sem0)

    compute_chunk(c + 1, gbuf1, o1_v)


def _cp(tc_tiling):
  cp = pltpu.CompilerParams()
  fields = pltpu.CompilerParams.__dataclass_fields__
  if "needs_layout_passes" in fields:
    cp = dataclasses.replace(cp, needs_layout_passes=False)
  if "use_tc_tiling_on_sc" in fields:
    cp = dataclasses.replace(cp, use_tc_tiling_on_sc=tc_tiling)
  return cp


@jax.jit
def _run(entity_embedding, relation_embedding, aidx, ridx, pidx, nidx):
  tr = pl.kernel(
      _tr_body,
      out_type=[jax.ShapeDtypeStruct((_NENT * _L,), jnp.float32)
                for _ in range(_NV)],
      mesh=plsc.VectorSubcoreMesh(core_axis_name="c", subcore_axis_name="s"),
      compiler_params=_cp(True),
      scratch_types=(
          [pltpu.VMEM((_D, _TCH), jnp.float32) for _ in range(2)]
          + [pltpu.VMEM((_TCH * _L,), jnp.float32) for _ in range(2 * _NV)]
          + [pltpu.VMEM((_D, _TTAIL), jnp.float32)]
          + [pltpu.VMEM((_TTAIL * _L,), jnp.float32) for _ in range(_NV)]
          + [pltpu.SemaphoreType.DMA for _ in range(4)]
      ),
  )
  flat_slabs = tr(entity_embedding.T)
  slabs = [s.reshape(_NENT, _L) for s in flat_slabs]

  run = pl.kernel(
      _sc_body,
      out_type=[
          jax.ShapeDtypeStruct((_B,), jnp.float32),
          jax.ShapeDtypeStruct((_B * _NEG,), jnp.float32),
      ],
      mesh=plsc.VectorSubcoreMesh(core_axis_name="c", subcore_axis_name="s"),
      compiler_params=_cp(False),
      scratch_types=(
          [
              pltpu.VMEM((_BPW,), jnp.int32),
              pltpu.VMEM((_BPW,), jnp.int32),
              pltpu.VMEM((_BPW,), jnp.int32),
          ]
          + [pltpu.VMEM((_BPW, _L), jnp.float32) for _ in range(_NV)]
          + [pltpu.VMEM((_BPW, _L), jnp.float32) for _ in range(_NV)]
          + [
              pltpu.VMEM((_BPW, _D), jnp.float32),
              pltpu.VMEM((_BPW, _D), jnp.float32),
              pltpu.VMEM((_BPW,), jnp.float32),
              pltpu.VMEM((_CB, _NEG), jnp.int32),
              pltpu.VMEM((_CB, _NEG), jnp.int32),
          ]
          + [pltpu.VMEM((_CROWS, _L), jnp.float32) for _ in range(2 * _NV)]
          + [
              pltpu.VMEM((_CROWS,), jnp.float32),
              pltpu.VMEM((_CROWS,), jnp.float32),
              pltpu.SemaphoreType.DMA,
              pltpu.SemaphoreType.DMA,
              pltpu.SemaphoreType.DMA,
              pltpu.SemaphoreType.DMA,
              pltpu.SemaphoreType.DMA,
          ]
      ),
  )
  return run(slabs[0], slabs[1], slabs[2], slabs[3],
             relation_embedding, aidx, ridx, pidx, nidx)


def kernel(entity_embedding, relation_embedding, subsampling_weight,
           positive_sample, negative_sample, queries):
  aidx = queries[:, 0].astype(jnp.int32)
  ridx = queries[:, 1].astype(jnp.int32)
  pidx = positive_sample.astype(jnp.int32)
  nidx = negative_sample.astype(jnp.int32)
  pos_logit, neg_flat = _run(
      entity_embedding, relation_embedding, aidx, ridx, pidx, nidx)
  return pos_logit, neg_flat.reshape(_B, _NEG), subsampling_weight


# speedup vs baseline: 1.0187x; 1.0187x over previous
import dataclasses
import functools

import jax
import jax.numpy as jnp
from jax import lax
from jax.experimental import pallas as pl
from jax.experimental.pallas import tpu as pltpu
from jax.experimental.pallas import tpu_sc as plsc

_GAMMA = 24.0
_B = 4096
_NEG = 128
_D = 64
_L = 16
_NV = _D // _L
_NC = 2
_NS = 16
_NW = _NC * _NS
_BPW = _B // _NW
_CB = 4
_CROWS = _CB * _NEG
_NCHUNK = _BPW // _CB

_NENT = 1000000
_TCH = 384
_TNCH = 2604
_TTAIL = _NENT - _TNCH * _TCH
_TITER = 82


def _tr_body(et_hbm, s0, s1, s2, s3,
             t0, t1, ob00, ob01, ob02, ob03, ob10, ob11, ob12, ob13,
             tt, obt0, obt1, obt2, obt3,
             semi0, semi1, semw0, semw1):
  outs = (s0, s1, s2, s3)
  obt = (obt0, obt1, obt2, obt3)
  obs = ((ob00, ob01, ob02, ob03), (ob10, ob11, ob12, ob13))
  tb = (t0, t1)
  semi = (semi0, semi1)
  semw = (semw0, semw1)
  wid = lax.axis_index("s") * _NC + lax.axis_index("c")
  lane = lax.iota(jnp.int32, _L)
  lane16 = lane * _L

  def start_in(c, par):
    @pl.when(c < _TNCH)
    def _():
      pltpu.async_copy(et_hbm.at[:, pl.ds(c * _TCH, _TCH)], tb[par], semi[par])

  def wait_in(par):
    pltpu.make_async_copy(et_hbm.at[:, pl.ds(0, _TCH)], tb[par], semi[par]).wait()

  def start_out(c, par):
    for m in range(_NV):
      pltpu.async_copy(obs[par][m],
                       outs[m].at[pl.ds(c * _TCH * _L, _TCH * _L)], semw[par])

  def wait_out(par):
    for m in range(_NV):
      pltpu.make_async_copy(obs[par][m],
                            outs[m].at[pl.ds(0, _TCH * _L)], semw[par]).wait()

  def transpose_chunk(tbuf, obufs, width):
    @pl.loop(0, width, step=_L)
    def _(e0):
      base = e0 * _L
      idxs = [lane16 + (base + i) for i in range(_L)]
      vs = [tbuf[i, pl.ds(e0, _L)] for i in range(_L)]
      for m in range(_NV):
        nxt = []
        for i in range(_L):
          if m + 1 < _NV:
            nxt.append(tbuf[(m + 1) * _L + i, pl.ds(e0, _L)])
          plsc.store_scatter(obufs[m], [idxs[i]], vs[i])
        vs = nxt

  start_in(wid, 0)
  start_in(wid + _NW, 1)

  @pl.loop(0, _TITER, step=2)
  def _(k):
    c0 = wid + k * _NW
    c1 = wid + (k + 1) * _NW

    @pl.when(c0 < _TNCH)
    def _():
      wait_in(0)

      @pl.when(k > 0)
      def _():
        wait_out(0)

      transpose_chunk(tb[0], obs[0], _TCH)
      start_in(c0 + 2 * _NW, 0)
      start_out(c0, 0)

    @pl.when(c1 < _TNCH)
    def _():
      wait_in(1)

      @pl.when(k > 0)
      def _():
        wait_out(1)

      transpose_chunk(tb[1], obs[1], _TCH)
      start_in(c1 + 2 * _NW, 1)
      start_out(c1, 1)

  @pl.when(wid + (_TITER - 2) * _NW < _TNCH)
  def _():
    wait_out(0)

  @pl.when(wid + (_TITER - 1) * _NW < _TNCH)
  def _():
    wait_out(1)

  @pl.when(wid == 0)
  def _():
    pltpu.async_copy(et_hbm.at[:, pl.ds(_TNCH * _TCH, _TTAIL)], tt,
                     semi0).wait()

    @pl.loop(0, _TTAIL, step=_L)
    def _(e0):
      base = e0 * _L
      idxs = [lane16 + (base + i) for i in range(_L)]
      vs = [tt[i, pl.ds(e0, _L)] for i in range(_L)]
      for m in range(_NV):
        nxt = []
        for i in range(_L):
          if m + 1 < _NV:
            nxt.append(tt[(m + 1) * _L + i, pl.ds(e0, _L)])
          plsc.store_scatter(obt[m], [idxs[i]], vs[i])
        vs = nxt

    for m in range(_NV):
      pltpu.sync_copy(obt[m],
                      outs[m].at[pl.ds(_TNCH * _TCH * _L, _TTAIL * _L)])


def _sc_body(e0, e1, e2, e3, rel_hbm, aidx_hbm, ridx_hbm, pidx_hbm, nidx_hbm,
             pos_hbm, neg_hbm,
             aidx_v, ridx_v, pidx_v,
             a0, a1, a2, a3, p0, p1, p2, p3, rrow_v, cent_v, pos_v,
             nidx0_v, nidx1_v,
             g00, g01, g02, g03, g10, g11, g12, g13, o0_v, o1_v,
             sem_a, sem_r, sem_p, sem0, sem1):
  slabs = (e0, e1, e2, e3)
  arow = (a0, a1, a2, a3)
  prow = (p0, p1, p2, p3)
  gbuf0 = (g00, g01, g02, g03)
  gbuf1 = (g10, g11, g12, g13)

  wid = lax.axis_index("s") * _NC + lax.axis_index("c")
  base = wid * _BPW
  nbase = wid * (_BPW * _NEG)

  pltpu.sync_copy(aidx_hbm.at[pl.ds(base, _BPW)], aidx_v)
  pltpu.sync_copy(ridx_hbm.at[pl.ds(base, _BPW)], ridx_v)
  pltpu.sync_copy(pidx_hbm.at[pl.ds(base, _BPW)], pidx_v)
  for m in range(_NV):
    pltpu.async_copy(slabs[m].at[aidx_v], arow[m], sem_a)
    pltpu.async_copy(slabs[m].at[pidx_v], prow[m], sem_p)
  cr = pltpu.async_copy(rel_hbm.at[ridx_v], rrow_v, sem_r)

  def start_gather(cidx, idxbuf, gbufs, sem):
    row_lo = base + cidx * _CB
    pltpu.sync_copy(nidx_hbm.at[pl.ds(row_lo, _CB)], idxbuf)
    for q in range(_CB):
      for m in range(_NV):
        pltpu.async_copy(slabs[m].at[idxbuf.at[q]],
                         gbufs[m].at[pl.ds(q * _NEG, _NEG)], sem)

  def wait_gather(idxbuf, gbufs, sem):
    for q in range(_CB):
      for m in range(_NV):
        pltpu.make_async_copy(slabs[m].at[idxbuf.at[q]],
                              gbufs[m].at[pl.ds(q * _NEG, _NEG)], sem).wait()

  start_gather(0, nidx0_v, gbuf0, sem0)

  for m in range(_NV):
    pltpu.make_async_copy(slabs[m].at[aidx_v], arow[m], sem_a).wait()
    pltpu.make_async_copy(slabs[m].at[pidx_v], prow[m], sem_p).wait()
  cr.wait()

  lane = lax.iota(jnp.int32, _L)

  @pl.loop(0, _BPW // _L)
  def _(g):
    out = jnp.zeros((_L,), jnp.float32)
    for jj in range(_L):
      b = g * _L + jj
      acc = None
      for k in range(_NV):
        c = arow[k][b, pl.ds(0, _L)] + rrow_v[b, pl.ds(k * _L, _L)]
        cent_v[b, pl.ds(k * _L, _L)] = c
        d = jnp.abs(c - prow[k][b, pl.ds(0, _L)])
        acc = d if acc is None else acc + d
      out = jnp.where(lane == jj, jnp.sum(acc), out)
    pos_v[pl.ds(g * _L, _L)] = _GAMMA - out

  pltpu.sync_copy(pos_v, pos_hbm.at[pl.ds(base, _BPW)])

  lane2 = lax.iota(jnp.int32, _L)

  def compute_chunk(cidx, gbufs, obuf):
    for bb in range(_CB):
      brow = cidx * _CB + bb
      cvec = [cent_v[brow, pl.ds(k * _L, _L)] for k in range(_NV)]

      @pl.loop(0, _NEG // _L)
      def _(g):
        out = jnp.zeros((_L,), jnp.float32)
        for jj in range(_L):
          r = bb * _NEG + g * _L + jj
          acc = jnp.abs(cvec[0] - gbufs[0][r, pl.ds(0, _L)])
          for k in range(1, _NV):
            acc = acc + jnp.abs(cvec[k] - gbufs[k][r, pl.ds(0, _L)])
          out = jnp.where(lane2 == jj, jnp.sum(acc), out)
        obuf[pl.ds(bb * _NEG + g * _L, _L)] = _GAMMA - out

    pltpu.sync_copy(obuf, neg_hbm.at[pl.ds(nbase + cidx * _CROWS, _CROWS)])

  @pl.loop(0, _NCHUNK, step=2)
  def _(c):
    wait_gather(nidx0_v, gbuf0, sem0)
    start_gather(c + 1, nidx1_v, gbuf1, sem1)
    compute_chunk(c, gbuf0, o0_v)

    wait_gather(nidx1_v, gbuf1, sem1)

    @pl.when(c + 2 < _NCHUNK)
    def _():
      start_gather(c + 2, nidx0_v, gbuf0, sem0)

    compute_chunk(c + 1, gbuf1, o1_v)


def _cp(tc_tiling):
  cp = pltpu.CompilerParams()
  fields = pltpu.CompilerParams.__dataclass_fields__
  if "needs_layout_passes" in fields:
    cp = dataclasses.replace(cp, needs_layout_passes=False)
  if "use_tc_tiling_on_sc" in fields:
    cp = dataclasses.replace(cp, use_tc_tiling_on_sc=tc_tiling)
  return cp


@jax.jit
def _run(entity_embedding, relation_embedding, aidx, ridx, pidx, nidx):
  tr = pl.kernel(
      _tr_body,
      out_type=[jax.ShapeDtypeStruct((_NENT * _L,), jnp.float32)
                for _ in range(_NV)],
      mesh=plsc.VectorSubcoreMesh(core_axis_name="c", subcore_axis_name="s"),
      compiler_params=_cp(True),
      scratch_types=(
          [pltpu.VMEM((_D, _TCH), jnp.float32) for _ in range(2)]
          + [pltpu.VMEM((_TCH * _L,), jnp.float32) for _ in range(2 * _NV)]
          + [pltpu.VMEM((_D, _TTAIL), jnp.float32)]
          + [pltpu.VMEM((_TTAIL * _L,), jnp.float32) for _ in range(_NV)]
          + [pltpu.SemaphoreType.DMA for _ in range(4)]
      ),
  )
  flat_slabs = tr(entity_embedding.T)
  slabs = [s.reshape(_NENT, _L) for s in flat_slabs]

  run = pl.kernel(
      _sc_body,
      out_type=[
          jax.ShapeDtypeStruct((_B,), jnp.float32),
          jax.ShapeDtypeStruct((_B * _NEG,), jnp.float32),
      ],
      mesh=plsc.VectorSubcoreMesh(core_axis_name="c", subcore_axis_name="s"),
      compiler_params=_cp(False),
      scratch_types=(
          [
              pltpu.VMEM((_BPW,), jnp.int32),
              pltpu.VMEM((_BPW,), jnp.int32),
              pltpu.VMEM((_BPW,), jnp.int32),
          ]
          + [pltpu.VMEM((_BPW, _L), jnp.float32) for _ in range(_NV)]
          + [pltpu.VMEM((_BPW, _L), jnp.float32) for _ in range(_NV)]
          + [
              pltpu.VMEM((_BPW, _D), jnp.float32),
              pltpu.VMEM((_BPW, _D), jnp.float32),
              pltpu.VMEM((_BPW,), jnp.float32),
              pltpu.VMEM((_CB, _NEG), jnp.int32),
              pltpu.VMEM((_CB, _NEG), jnp.int32),
          ]
          + [pltpu.VMEM((_CROWS, _L), jnp.float32) for _ in range(2 * _NV)]
          + [
              pltpu.VMEM((_CROWS,), jnp.float32),
              pltpu.VMEM((_CROWS,), jnp.float32),
              pltpu.SemaphoreType.DMA,
              pltpu.SemaphoreType.DMA,
              pltpu.SemaphoreType.DMA,
              pltpu.SemaphoreType.DMA,
              pltpu.SemaphoreType.DMA,
          ]
      ),
  )
  return run(slabs[0], slabs[1], slabs[2], slabs[3],
             relation_embedding, aidx, ridx, pidx, nidx)


def kernel(entity_embedding, relation_embedding, subsampling_weight,
           positive_sample, negative_sample, queries):
  aidx = queries[:, 0].astype(jnp.int32)
  ridx = queries[:, 1].astype(jnp.int32)
  pidx = positive_sample.astype(jnp.int32)
  nidx = negative_sample.astype(jnp.int32)
  pos_logit, neg_flat = _run(
      entity_embedding, relation_embedding, aidx, ridx, pidx, nidx)
  return pos_logit, neg_flat.reshape(_B, _NEG), subsampling_weight
